# baseline (speedup 1.0000x reference)
import functools

import jax
import jax.numpy as jnp
from jax.experimental import pallas as pl

T = 16384
D = 2048
E = 8
K = 2
BLK = 512


def _top2(get_col, n):
    m1 = get_col(0)
    shape = m1.shape
    i1 = jnp.zeros(shape, jnp.int32)
    m2 = jnp.full(shape, -jnp.inf, jnp.float32)
    i2 = jnp.zeros(shape, jnp.int32)
    for e in range(1, n):
        l = get_col(e)
        ev = jnp.full(shape, e, jnp.int32)
        gt1 = l > m1
        gt2 = l > m2
        m2 = jnp.where(gt1, m1, jnp.where(gt2, l, m2))
        i2 = jnp.where(gt1, i1, jnp.where(gt2, ev, i2))
        m1 = jnp.where(gt1, l, m1)
        i1 = jnp.where(gt1, ev, i1)
    return m1, i1, m2, i2


def _fused_body(h_ref, w_ref, vals_ref, idx_ref):
    logits = jax.lax.dot_general(
        h_ref[...], w_ref[...],
        (((1,), (1,)), ((), ())),
        preferred_element_type=jnp.float32,
    )
    m1, i1, m2, i2 = _top2(lambda e: logits[:, e:e + 1], E)
    w1 = 1.0 / (1.0 + jnp.exp(m2 - m1))
    vals_ref[:, 0:1] = w1
    vals_ref[:, 1:2] = 1.0 - w1
    idx_ref[:, 0:1] = i1
    idx_ref[:, 1:2] = i2


@jax.jit
def kernel(hidden_states, W_gate):
    grid = (T // BLK,)
    vals, idx = pl.pallas_call(
        _fused_body,
        grid=grid,
        in_specs=[
            pl.BlockSpec((BLK, D), lambda i: (i, 0)),
            pl.BlockSpec((E, D), lambda i: (0, 0)),
        ],
        out_specs=[
            pl.BlockSpec((BLK, K), lambda i: (i, 0)),
            pl.BlockSpec((BLK, K), lambda i: (i, 0)),
        ],
        out_shape=[
            jax.ShapeDtypeStruct((T, K), jnp.float32),
            jax.ShapeDtypeStruct((T, K), jnp.int32),
        ],
    )(hidden_states, W_gate)
    return vals, idx


# speedup vs baseline: 1.4458x; 1.4458x over previous
import functools

import jax
import jax.numpy as jnp
from jax import lax
from jax.experimental import pallas as pl
from jax.experimental.pallas import tpu as pltpu
from jax.experimental.pallas import tpu_sc as plsc

T = 16384
D = 2048
E = 8
K = 2

NC, NS, L = 2, 16, 16
NW = NC * NS
CHUNK = T // NW
NG = CHUNK // L


def _mm_body(h_ref, w_ref, o_ref):
    o_ref[0] = lax.dot_general(
        w_ref[...], h_ref[...],
        (((1,), (1,)), ((), ())),
        preferred_element_type=jnp.float32,
    )


def _logits_tc(hidden_states, W_gate):
    return pl.pallas_call(
        _mm_body,
        grid=(NW,),
        in_specs=[
            pl.BlockSpec((CHUNK, D), lambda i: (i, 0)),
            pl.BlockSpec((E, D), lambda i: (0, 0)),
        ],
        out_specs=pl.BlockSpec((1, E, CHUNK), lambda i: (i, 0, 0)),
        out_shape=jax.ShapeDtypeStruct((NW, E, CHUNK), jnp.float32),
    )(hidden_states, W_gate)


def _route_body(logits_hbm, vals_hbm, idx_hbm, lv, wv, iv):
    wid = lax.axis_index("s") * NC + lax.axis_index("c")
    base = wid * CHUNK
    pltpu.sync_copy(logits_hbm.at[wid], lv)
    for g in range(NG):
        sl = pl.ds(g * L, L)
        m1 = lv[0, sl]
        i1 = jnp.zeros((L,), jnp.int32)
        m2 = jnp.full((L,), -jnp.inf, jnp.float32)
        i2 = jnp.zeros((L,), jnp.int32)
        for e in range(1, E):
            l = lv[e, sl]
            ev = jnp.full((L,), e, jnp.int32)
            gt1 = l > m1
            gt2 = l > m2
            m2 = jnp.where(gt1, m1, jnp.where(gt2, l, m2))
            i2 = jnp.where(gt1, i1, jnp.where(gt2, ev, i2))
            m1 = jnp.where(gt1, l, m1)
            i1 = jnp.where(gt1, ev, i1)
        w1 = 1.0 / (1.0 + jnp.exp(m2 - m1))
        wv[0, sl] = w1
        wv[1, sl] = 1.0 - w1
        iv[0, sl] = i1
        iv[1, sl] = i2
    pltpu.sync_copy(wv, vals_hbm.at[:, pl.ds(base, CHUNK)])
    pltpu.sync_copy(iv, idx_hbm.at[:, pl.ds(base, CHUNK)])


@functools.lru_cache(maxsize=None)
def _make_route_sc():
    return pl.kernel(
        _route_body,
        mesh=plsc.VectorSubcoreMesh(
            core_axis_name="c", subcore_axis_name="s",
            num_cores=NC, num_subcores=NS,
        ),
        out_type=[
            jax.ShapeDtypeStruct((K, T), jnp.float32),
            jax.ShapeDtypeStruct((K, T), jnp.int32),
        ],
        scratch_types=[
            pltpu.VMEM((E, CHUNK), jnp.float32),
            pltpu.VMEM((K, CHUNK), jnp.float32),
            pltpu.VMEM((K, CHUNK), jnp.int32),
        ],
    )


@jax.jit
def kernel(hidden_states, W_gate):
    logits = _logits_tc(hidden_states, W_gate)
    vals, idx = _make_route_sc()(logits)
    return vals.T, idx.T


# speedup vs baseline: 1.5530x; 1.0742x over previous
import functools

import jax
import jax.numpy as jnp
from jax import lax
from jax.experimental import pallas as pl
from jax.experimental.pallas import tpu as pltpu
from jax.experimental.pallas import tpu_sc as plsc

T = 16384
D = 2048
E = 8
K = 2

NC, NS, L = 2, 16, 16
NW = NC * NS
CHUNK = T // NW
NG = CHUNK // L


MMBLK = 2048


def _mm_body(h_ref, w_ref, o_ref):
    o_ref[...] = lax.dot_general(
        w_ref[...], h_ref[...],
        (((1,), (1,)), ((), ())),
        preferred_element_type=jnp.float32,
    )


def _logits_tc(hidden_states, W_gate):
    return pl.pallas_call(
        _mm_body,
        grid=(T // MMBLK,),
        in_specs=[
            pl.BlockSpec((MMBLK, D), lambda i: (i, 0)),
            pl.BlockSpec((E, D), lambda i: (0, 0)),
        ],
        out_specs=pl.BlockSpec((E, MMBLK), lambda i: (0, i)),
        out_shape=jax.ShapeDtypeStruct((E, T), jnp.float32),
    )(hidden_states, W_gate)


def _route_body(logits_hbm, vals_hbm, idx_hbm, lv, wv, iv):
    wid = lax.axis_index("s") * NC + lax.axis_index("c")
    base = wid * CHUNK
    pltpu.sync_copy(logits_hbm.at[:, pl.ds(base, CHUNK)], lv)
    for g in range(NG):
        sl = pl.ds(g * L, L)
        m1 = lv[0, sl]
        i1 = jnp.zeros((L,), jnp.int32)
        m2 = jnp.full((L,), -jnp.inf, jnp.float32)
        i2 = jnp.zeros((L,), jnp.int32)
        for e in range(1, E):
            l = lv[e, sl]
            ev = jnp.full((L,), e, jnp.int32)
            gt1 = l > m1
            gt2 = l > m2
            m2 = jnp.where(gt1, m1, jnp.where(gt2, l, m2))
            i2 = jnp.where(gt1, i1, jnp.where(gt2, ev, i2))
            m1 = jnp.where(gt1, l, m1)
            i1 = jnp.where(gt1, ev, i1)
        w1 = 1.0 / (1.0 + jnp.exp(m2 - m1))
        wv[0, sl] = w1
        wv[1, sl] = 1.0 - w1
        iv[0, sl] = i1
        iv[1, sl] = i2
    pltpu.sync_copy(wv, vals_hbm.at[:, pl.ds(base, CHUNK)])
    pltpu.sync_copy(iv, idx_hbm.at[:, pl.ds(base, CHUNK)])


@functools.lru_cache(maxsize=None)
def _make_route_sc():
    return pl.kernel(
        _route_body,
        mesh=plsc.VectorSubcoreMesh(
            core_axis_name="c", subcore_axis_name="s",
            num_cores=NC, num_subcores=NS,
        ),
        out_type=[
            jax.ShapeDtypeStruct((K, T), jnp.float32),
            jax.ShapeDtypeStruct((K, T), jnp.int32),
        ],
        scratch_types=[
            pltpu.VMEM((E, CHUNK), jnp.float32),
            pltpu.VMEM((K, CHUNK), jnp.float32),
            pltpu.VMEM((K, CHUNK), jnp.int32),
        ],
    )


@jax.jit
def kernel(hidden_states, W_gate):
    logits = _logits_tc(hidden_states, W_gate)
    vals, idx = _make_route_sc()(logits)
    return vals.T, idx.T


# speedup vs baseline: 1.5936x; 1.0261x over previous
import functools

import jax
import jax.numpy as jnp
from jax import lax
from jax.experimental import pallas as pl
from jax.experimental.pallas import tpu as pltpu
from jax.experimental.pallas import tpu_sc as plsc

T = 16384
D = 2048
E = 8
K = 2

NC, NS, L = 2, 16, 16
NW = NC * NS
CHUNK = T // NW
NG = CHUNK // L


MMBLK = 512
NSTEP = T // MMBLK
RING = 4


def _mm_body(h_hbm, w_ref, o_ref, bufs, sems):
    def start(i):
        pltpu.make_async_copy(
            h_hbm.at[pl.ds(i * MMBLK, MMBLK), :],
            bufs.at[i % RING],
            sems.at[i % RING],
        ).start()

    def wait(i):
        pltpu.make_async_copy(
            h_hbm.at[pl.ds(i * MMBLK, MMBLK), :],
            bufs.at[i % RING],
            sems.at[i % RING],
        ).wait()

    for i in range(RING - 1):
        start(i)
    for i in range(NSTEP):
        if i + RING - 1 < NSTEP:
            start(i + RING - 1)
        wait(i)
        o_ref[:, pl.ds(i * MMBLK, MMBLK)] = lax.dot_general(
            w_ref[...], bufs[i % RING],
            (((1,), (1,)), ((), ())),
            preferred_element_type=jnp.float32,
        )


def _logits_tc(hidden_states, W_gate):
    return pl.pallas_call(
        _mm_body,
        in_specs=[
            pl.BlockSpec(memory_space=pltpu.HBM),
            pl.BlockSpec(memory_space=pltpu.VMEM),
        ],
        out_specs=pl.BlockSpec(memory_space=pltpu.VMEM),
        out_shape=jax.ShapeDtypeStruct((E, T), jnp.float32),
        scratch_shapes=[
            pltpu.VMEM((RING, MMBLK, D), jnp.float32),
            pltpu.SemaphoreType.DMA((RING,)),
        ],
    )(hidden_states, W_gate)


def _route_body(logits_hbm, vals_hbm, idx_hbm, lv, wv, iv):
    wid = lax.axis_index("s") * NC + lax.axis_index("c")
    base = wid * CHUNK
    pltpu.sync_copy(logits_hbm.at[:, pl.ds(base, CHUNK)], lv)
    for g in range(NG):
        sl = pl.ds(g * L, L)
        m1 = lv[0, sl]
        i1 = jnp.zeros((L,), jnp.int32)
        m2 = jnp.full((L,), -jnp.inf, jnp.float32)
        i2 = jnp.zeros((L,), jnp.int32)
        for e in range(1, E):
            l = lv[e, sl]
            ev = jnp.full((L,), e, jnp.int32)
            gt1 = l > m1
            gt2 = l > m2
            m2 = jnp.where(gt1, m1, jnp.where(gt2, l, m2))
            i2 = jnp.where(gt1, i1, jnp.where(gt2, ev, i2))
            m1 = jnp.where(gt1, l, m1)
            i1 = jnp.where(gt1, ev, i1)
        w1 = 1.0 / (1.0 + jnp.exp(m2 - m1))
        wv[0, sl] = w1
        wv[1, sl] = 1.0 - w1
        iv[0, sl] = i1
        iv[1, sl] = i2
    pltpu.sync_copy(wv, vals_hbm.at[:, pl.ds(base, CHUNK)])
    pltpu.sync_copy(iv, idx_hbm.at[:, pl.ds(base, CHUNK)])


@functools.lru_cache(maxsize=None)
def _make_route_sc():
    return pl.kernel(
        _route_body,
        mesh=plsc.VectorSubcoreMesh(
            core_axis_name="c", subcore_axis_name="s",
            num_cores=NC, num_subcores=NS,
        ),
        out_type=[
            jax.ShapeDtypeStruct((K, T), jnp.float32),
            jax.ShapeDtypeStruct((K, T), jnp.int32),
        ],
        scratch_types=[
            pltpu.VMEM((E, CHUNK), jnp.float32),
            pltpu.VMEM((K, CHUNK), jnp.float32),
            pltpu.VMEM((K, CHUNK), jnp.int32),
        ],
    )


@jax.jit
def kernel(hidden_states, W_gate):
    logits = _logits_tc(hidden_states, W_gate)
    vals, idx = _make_route_sc()(logits)
    return vals.T, idx.T
